# baseline (speedup 1.0000x reference)
import jax
import jax.numpy as jnp
from jax import lax
from jax.experimental import pallas as pl
from jax.experimental.pallas import tpu as pltpu
from jax.experimental.pallas import tpu_sc as plsc

NUM_WORKERS = 32
ROWS = 4096
HIST = 200
D = 64
DPAD = 128
VOCAB = 1000000
NTILES = 7813
C = ROWS // NUM_WORKERS
TP = C + 1
NBUF = 4
LEAD = 3
NT = 2
OP = DPAD + 1

assert HIST % NBUF == 0 and LEAD < NBUF


def _repack(wt_hbm, out_hbm, inb, outb, sem_g, sem_w):
    wid = lax.axis_index("s") * 2 + lax.axis_index("c")
    nfull = NTILES - 1
    nb = nfull // NUM_WORKERS
    ext = nfull - nb * NUM_WORKERS
    lo = wid * nb + jnp.minimum(wid, ext)
    cnt = nb + jnp.where(wid < ext, 1, 0)

    iota = lax.iota(jnp.int32, 16)

    def fetch(T, slot):
        for k in range(8):
            pltpu.async_copy(wt_hbm.at[pl.ds(8 * k, 8), pl.ds(T * 128, 128)],
                             inb.at[slot, k], sem_g)

    def wait_fetch():
        for k in range(8):
            pltpu.make_async_copy(wt_hbm.at[pl.ds(0, 8), pl.ds(0, 128)],
                                  inb.at[0, 0], sem_g).wait()

    def transpose(slot, oslot):
        def tr(ks, carry):
            k = ks // 8
            s = ks % 8
            c0 = jnp.full((16,), 8 * k + s, jnp.int32)
            for m in range(8):
                x = inb[slot, k, s, pl.ds(16 * m, 16)]
                li = iota + 16 * m
                plsc.store_scatter(outb.at[oslot], [li, c0], x)
                plsc.store_scatter(outb.at[oslot], [li, c0 + D], x)
            return carry

        lax.fori_loop(0, 64, tr, 0, unroll=4)

    def writeback(T, oslot):
        pltpu.async_copy(outb.at[oslot, :, 0:DPAD],
                         out_hbm.at[pl.ds(T * 128, 128)], sem_w)

    def wait_wb():
        pltpu.make_async_copy(outb.at[0, :, 0:DPAD],
                              out_hbm.at[pl.ds(0, 128)], sem_w).wait()

    fetch(lo, 0)

    def step(i, carry):
        T = lo + i
        slot = lax.rem(i, 2)

        @pl.when(i + 1 < cnt)
        def _():
            fetch(T + 1, 1 - slot)

        wait_fetch()

        @pl.when(i >= 2)
        def _():
            wait_wb()

        transpose(slot, slot)
        writeback(T, slot)
        return carry

    lax.fori_loop(0, cnt, step, 0)
    wait_wb()
    wait_wb()


def _gather_body(ids_hbm, table_hbm, out_hbm, idx_v, bufs, tbufs, sem_g, sem_w):
    wid = lax.axis_index("s") * 2 + lax.axis_index("c")
    base = wid * C

    pltpu.sync_copy(ids_hbm.at[:, pl.ds(base, C)], idx_v)

    def gather(j, b):
        pltpu.async_copy(table_hbm.at[idx_v.at[j]], bufs.at[b], sem_g)

    def writeback(j, t):
        pltpu.async_copy(tbufs.at[t, :, :, 0:C],
                         out_hbm.at[j, :, wid, :, :], sem_w)

    def wait_g():
        pltpu.make_async_copy(table_hbm.at[idx_v.at[0]], bufs.at[0],
                              sem_g).wait()

    def wait_w():
        pltpu.make_async_copy(tbufs.at[0, :, :, 0:C],
                              out_hbm.at[0, :, wid, :, :], sem_w).wait()

    iota = lax.iota(jnp.int32, 16)
    kvecs = [(iota + 16 * k) // 8 for k in range(D // 16)]
    svecs = [(iota + 16 * k) % 8 for k in range(D // 16)]

    def transpose(b, t):
        def tr_row(r, carry):
            row_idx = jnp.full((16,), r, jnp.int32)
            for k in range(D // 16):
                x = bufs[b, r, pl.ds(16 * k, 16)]
                plsc.store_scatter(tbufs.at[t], [kvecs[k], svecs[k], row_idx],
                                   x)
            return carry

        lax.fori_loop(0, C, tr_row, 0, unroll=8)

    for i in range(LEAD):
        gather(i, i)
    for b in range(NBUF):
        gather(b + LEAD, (b + LEAD) % NBUF)
        wait_g()
        if b >= NT:
            wait_w()
        transpose(b, b % NT)
        writeback(b, b % NT)

    def step(g, carry):
        j0 = g * NBUF
        for b in range(NBUF):
            gather(j0 + b + LEAD, (b + LEAD) % NBUF)
            wait_g()
            wait_w()
            transpose(b, b % NT)
            writeback(j0 + b, b % NT)
        return carry

    lax.fori_loop(1, HIST // NBUF - 1, step, 0)

    j0 = HIST - NBUF
    for b in range(NBUF):
        if b < NBUF - LEAD:
            gather(j0 + b + LEAD, (b + LEAD) % NBUF)
        wait_g()
        wait_w()
        transpose(b, b % NT)
        writeback(j0 + b, b % NT)
    for _ in range(NT):
        wait_w()


def kernel(token_ids, weights):
    repack = pl.kernel(
        _repack,
        out_type=jax.ShapeDtypeStruct((VOCAB, DPAD), jnp.float32),
        mesh=plsc.VectorSubcoreMesh(core_axis_name="c", subcore_axis_name="s"),
        scratch_types=[
            pltpu.VMEM((2, 8, 8, 128), jnp.float32),
            pltpu.VMEM((2, 128, OP), jnp.float32),
            pltpu.SemaphoreType.DMA,
            pltpu.SemaphoreType.DMA,
        ],
        compiler_params=pltpu.CompilerParams(use_tc_tiling_on_sc=True,
                                             needs_layout_passes=False),
    )
    gather = pl.kernel(
        _gather_body,
        out_type=jax.ShapeDtypeStruct((HIST, D // 8, NUM_WORKERS, 8, C),
                                      jnp.float32),
        mesh=plsc.VectorSubcoreMesh(core_axis_name="c", subcore_axis_name="s"),
        scratch_types=[
            pltpu.VMEM((HIST, C), jnp.int32),
            pltpu.VMEM((NBUF, C, DPAD), jnp.float32),
            pltpu.VMEM((NT, D // 8, 8, TP), jnp.float32),
            pltpu.SemaphoreType.DMA,
            pltpu.SemaphoreType.DMA,
        ],
        compiler_params=pltpu.CompilerParams(use_tc_tiling_on_sc=False,
                                             needs_layout_passes=False),
    )
    wdup = repack(weights.T)
    tail = jnp.tile(weights[128 * (NTILES - 1):, :], (1, 2))
    wdup = lax.dynamic_update_slice(wdup, tail, (128 * (NTILES - 1), 0))
    ids_flat = lax.optimization_barrier(jnp.ravel(token_ids.T.astype(jnp.int32)))
    out5 = gather(ids_flat.reshape(HIST, ROWS), wdup)
    out = out5.transpose(2, 4, 0, 1, 3).reshape(ROWS, HIST, D)
    return out


# speedup vs baseline: 2.9737x; 2.9737x over previous
import jax
import jax.numpy as jnp
from jax import lax
from jax.experimental import pallas as pl
from jax.experimental.pallas import tpu as pltpu
from jax.experimental.pallas import tpu_sc as plsc

NUM_WORKERS = 32
ROWS = 4096
HIST = 200
D = 64
DPAD = 128
C = ROWS // NUM_WORKERS
TP = C + 1
NBUF = 4
LEAD = 3
NT = 2

assert HIST % NBUF == 0 and LEAD < NBUF


def _body(ids_hbm, table_hbm, out_hbm, idx_v, bufs, tbufs, sem_g, sem_w):
    wid = lax.axis_index("s") * 2 + lax.axis_index("c")
    base = wid * C

    pltpu.sync_copy(ids_hbm.at[:, pl.ds(base, C)], idx_v)

    def gather(j, b):
        pltpu.async_copy(table_hbm.at[idx_v.at[j]], bufs.at[b], sem_g)

    def writeback(j, t):
        pltpu.async_copy(tbufs.at[t, :, :, 0:C],
                         out_hbm.at[j, :, wid, :, :], sem_w)

    def wait_g():
        pltpu.make_async_copy(table_hbm.at[idx_v.at[0]], bufs.at[0],
                              sem_g).wait()

    def wait_w():
        pltpu.make_async_copy(tbufs.at[0, :, :, 0:C],
                              out_hbm.at[0, :, wid, :, :], sem_w).wait()

    iota = lax.iota(jnp.int32, 16)
    kvecs = [(iota + 16 * k) // 8 for k in range(D // 16)]
    svecs = [(iota + 16 * k) % 8 for k in range(D // 16)]

    def transpose(b, t):
        def tr_row(r, carry):
            row_idx = jnp.full((16,), r, jnp.int32)
            for k in range(D // 16):
                x = bufs[b, r, pl.ds(16 * k, 16)]
                plsc.store_scatter(tbufs.at[t], [kvecs[k], svecs[k], row_idx],
                                   x)
            return carry

        lax.fori_loop(0, C, tr_row, 0, unroll=8)

    for i in range(LEAD):
        gather(i, i)
    for b in range(NBUF):
        gather(b + LEAD, (b + LEAD) % NBUF)
        wait_g()
        if b >= NT:
            wait_w()
        transpose(b, b % NT)
        writeback(b, b % NT)

    def step(g, carry):
        j0 = g * NBUF
        for b in range(NBUF):
            gather(j0 + b + LEAD, (b + LEAD) % NBUF)
            wait_g()
            wait_w()
            transpose(b, b % NT)
            writeback(j0 + b, b % NT)
        return carry

    lax.fori_loop(1, HIST // NBUF - 1, step, 0)

    j0 = HIST - NBUF
    for b in range(NBUF):
        if b < NBUF - LEAD:
            gather(j0 + b + LEAD, (b + LEAD) % NBUF)
        wait_g()
        wait_w()
        transpose(b, b % NT)
        writeback(j0 + b, b % NT)
    for _ in range(NT):
        wait_w()


def kernel(token_ids, weights):
    run = pl.kernel(
        _body,
        out_type=jax.ShapeDtypeStruct((HIST, D // 8, NUM_WORKERS, 8, C),
                                      jnp.float32),
        mesh=plsc.VectorSubcoreMesh(core_axis_name="c", subcore_axis_name="s"),
        scratch_types=[
            pltpu.VMEM((HIST, C), jnp.int32),
            pltpu.VMEM((NBUF, C, D), jnp.float32),
            pltpu.VMEM((NT, D // 8, 8, TP), jnp.float32),
            pltpu.SemaphoreType.DMA,
            pltpu.SemaphoreType.DMA,
        ],
        compiler_params=pltpu.CompilerParams(use_tc_tiling_on_sc=False,
                                             needs_layout_passes=False),
    )
    wpad = jnp.pad(weights, ((0, 0), (0, DPAD - D))).reshape(-1, D)
    ids_flat = lax.optimization_barrier(
        jnp.ravel(token_ids.T.astype(jnp.int32)) * 2)
    out5 = run(ids_flat.reshape(HIST, ROWS), wpad)
    out = out5.transpose(2, 4, 0, 1, 3).reshape(ROWS, HIST, D)
    return out


# speedup vs baseline: 4.0108x; 1.3488x over previous
import jax
import jax.numpy as jnp
from jax import lax
from jax.experimental import pallas as pl
from jax.experimental.pallas import tpu as pltpu
from jax.experimental.pallas import tpu_sc as plsc

NUM_WORKERS = 32
ROWS = 4096
HIST = 200
D = 64
DPAD = 128
C = ROWS // NUM_WORKERS
TP = C + 1
NBUF = 4
LEAD = 3
NT = 2

assert HIST % NBUF == 0 and LEAD < NBUF


def _body(ids_hbm, table_hbm, out_hbm, idx_v, bufs, tbufs, sem_g, sem_w):
    wid = lax.axis_index("s") * 2 + lax.axis_index("c")
    base = wid * C

    pltpu.sync_copy(ids_hbm.at[:, pl.ds(base, C)], idx_v)

    def gather(j, b):
        pltpu.async_copy(table_hbm.at[idx_v.at[j]], bufs.at[b], sem_g)

    def writeback(j, t):
        pltpu.async_copy(tbufs.at[t, :, :, 0:C],
                         out_hbm.at[j, :, wid, :, :], sem_w)

    def wait_g():
        pltpu.make_async_copy(table_hbm.at[idx_v.at[0]], bufs.at[0],
                              sem_g).wait()

    def wait_w():
        pltpu.make_async_copy(tbufs.at[0, :, :, 0:C],
                              out_hbm.at[0, :, wid, :, :], sem_w).wait()

    iota = lax.iota(jnp.int32, 16)
    kvecs = [(iota + 16 * k) // 8 for k in range(D // 16)]
    svecs = [(iota + 16 * k) % 8 for k in range(D // 16)]

    def transpose(b, t):
        @plsc.parallel_loop(0, C, unroll=8)
        def tr_row(r):
            row_idx = jnp.full((16,), r, jnp.int32)
            for k in range(D // 16):
                x = bufs[b, r, pl.ds(16 * k, 16)]
                plsc.store_scatter(tbufs.at[t], [kvecs[k], svecs[k], row_idx],
                                   x)

    for i in range(LEAD):
        gather(i, i)
    for b in range(NBUF):
        gather(b + LEAD, (b + LEAD) % NBUF)
        wait_g()
        if b >= NT:
            wait_w()
        transpose(b, b % NT)
        writeback(b, b % NT)

    def step(g, carry):
        j0 = g * NBUF
        for b in range(NBUF):
            gather(j0 + b + LEAD, (b + LEAD) % NBUF)
            wait_g()
            wait_w()
            transpose(b, b % NT)
            writeback(j0 + b, b % NT)
        return carry

    lax.fori_loop(1, HIST // NBUF - 1, step, 0)

    j0 = HIST - NBUF
    for b in range(NBUF):
        if b < NBUF - LEAD:
            gather(j0 + b + LEAD, (b + LEAD) % NBUF)
        wait_g()
        wait_w()
        transpose(b, b % NT)
        writeback(j0 + b, b % NT)
    for _ in range(NT):
        wait_w()


def kernel(token_ids, weights):
    run = pl.kernel(
        _body,
        out_type=jax.ShapeDtypeStruct((HIST, D // 8, NUM_WORKERS, 8, C),
                                      jnp.float32),
        mesh=plsc.VectorSubcoreMesh(core_axis_name="c", subcore_axis_name="s"),
        scratch_types=[
            pltpu.VMEM((HIST, C), jnp.int32),
            pltpu.VMEM((NBUF, C, D), jnp.float32),
            pltpu.VMEM((NT, D // 8, 8, TP), jnp.float32),
            pltpu.SemaphoreType.DMA,
            pltpu.SemaphoreType.DMA,
        ],
        compiler_params=pltpu.CompilerParams(use_tc_tiling_on_sc=False,
                                             needs_layout_passes=False),
    )
    wpad = jnp.pad(weights, ((0, 0), (0, DPAD - D))).reshape(-1, D)
    ids_flat = lax.optimization_barrier(
        jnp.ravel(token_ids.T.astype(jnp.int32)) * 2)
    out5 = run(ids_flat.reshape(HIST, ROWS), wpad)
    out = out5.transpose(2, 4, 0, 1, 3).reshape(ROWS, HIST, D)
    return out
